# baseline (speedup 1.0000x reference)
import functools

import jax
import jax.numpy as jnp
from jax import lax
from jax.experimental import pallas as pl
from jax.experimental.pallas import tpu as pltpu
from jax.experimental.pallas import tpu_sc as plsc

NC = 2
NS = 16
NW = NC * NS
LN = 16
ROW = 128


def _mesh():
  return plsc.VectorSubcoreMesh(
      core_axis_name="c", subcore_axis_name="s",
      num_cores=NC, num_subcores=NS)


_PARAMS = pltpu.CompilerParams(use_tc_tiling_on_sc=False,
                               needs_layout_passes=False)


def _cdiv(a, b):
  return (a + b - 1) // b


def _fill(ref, n, value):
  v = jnp.full((LN,), value, ref.dtype)

  def body(i, _):
    ref[pl.ds(i * LN, LN)] = v
    return 0

  lax.fori_loop(0, n // LN, body, 0)


def _fill2d(ref, rows, cols, value):
  v = jnp.full((LN,), value, ref.dtype)

  def body(i, _):
    j = i // (cols // LN)
    o = (i % (cols // LN)) * LN
    ref[j, pl.ds(o, LN)] = v
    return 0

  lax.fori_loop(0, rows * (cols // LN), body, 0)


@functools.cache
def _degree_kernel(R, KB, NPAD):
  nchunk = R // KB
  rounds = _cdiv(nchunk, NW)
  sl = NPAD // NS

  def body(dst2d, out, acc, idx, ones, zbuf, sems):
    c = lax.axis_index("c")
    s = lax.axis_index("s")
    w32 = c * NS + s
    _fill(ones, ROW, 1.0)
    _fill(zbuf, sl, 0.0)
    pltpu.sync_copy(zbuf, acc.at[pl.ds(s * sl, sl)])
    plsc.subcore_barrier()

    def round_body(k, _):
      cid = w32 + k * NW

      @pl.when(cid < nchunk)
      def _():
        pltpu.sync_copy(dst2d.at[pl.ds(cid * KB, KB)], idx)
        descs = [pltpu.async_copy(ones, acc.at[idx.at[j]], sems, add=True)
                 for j in range(KB)]
        for d in descs:
          d.wait()
      return 0

    lax.fori_loop(0, rounds, round_body, 0)
    plsc.subcore_barrier()
    pltpu.sync_copy(acc.at[pl.ds(s * sl, sl)], zbuf)
    pltpu.sync_copy(zbuf, out.at[pl.ds(c * NPAD + s * sl, sl)])

  return pl.kernel(
      body,
      out_type=jax.ShapeDtypeStruct((NC * NPAD,), jnp.float32),
      mesh=_mesh(),
      compiler_params=_PARAMS,
      scratch_types=[
          pltpu.VMEM_SHARED((NPAD,), jnp.float32),
          pltpu.VMEM((KB, ROW), jnp.int32),
          pltpu.VMEM((ROW,), jnp.float32),
          pltpu.VMEM((sl,), jnp.float32),
          pltpu.SemaphoreType.DMA,
      ],
  )


@functools.cache
def _conv_kernel(w, R, KB, NPAD):
  nchunk = R // KB
  rounds = _cdiv(nchunk, NW)
  sl = NPAD // NS

  def body(src2d, dst2d, *rest):
    us = rest[:w]
    outs = rest[w:2 * w]
    utab = rest[2 * w:3 * w]
    acc = rest[3 * w:4 * w]
    idxs, idxd, val, zbuf, semg, sems = rest[4 * w:4 * w + 6]
    c = lax.axis_index("c")
    s = lax.axis_index("s")
    w32 = c * NS + s
    _fill(zbuf, sl, 0.0)
    tsl = pl.ds(s * sl, sl)
    for cc in range(w):
      pltpu.sync_copy(zbuf, acc[cc].at[tsl])
    for cc in range(w):
      pltpu.sync_copy(us[cc].at[tsl], zbuf)
      pltpu.sync_copy(zbuf, utab[cc].at[tsl])
    plsc.subcore_barrier()

    def round_body(k, _):
      cid = w32 + k * NW

      @pl.when(cid < nchunk)
      def _():
        csl = pl.ds(cid * KB, KB)
        pltpu.sync_copy(src2d.at[csl], idxs)
        pltpu.sync_copy(dst2d.at[csl], idxd)
        descs = [
            pltpu.async_copy(utab[cc].at[idxs.at[j]], val.at[cc * KB + j],
                             semg)
            for j in range(KB) for cc in range(w)]
        for d in descs:
          d.wait()
        descs = [
            pltpu.async_copy(val.at[cc * KB + j], acc[cc].at[idxd.at[j]],
                             sems, add=True)
            for j in range(KB) for cc in range(w)]
        for d in descs:
          d.wait()
      return 0

    lax.fori_loop(0, rounds, round_body, 0)
    plsc.subcore_barrier()
    osl = pl.ds(c * NPAD + s * sl, sl)
    for cc in range(w):
      pltpu.sync_copy(acc[cc].at[tsl], zbuf)
      pltpu.sync_copy(zbuf, outs[cc].at[osl])

  return pl.kernel(
      body,
      out_type=[jax.ShapeDtypeStruct((NC * NPAD,), jnp.float32)] * w,
      mesh=_mesh(),
      compiler_params=_PARAMS,
      scratch_types=(
          [pltpu.VMEM_SHARED((NPAD,), jnp.float32)] * (2 * w) + [
              pltpu.VMEM((KB, ROW), jnp.int32),
              pltpu.VMEM((KB, ROW), jnp.int32),
              pltpu.VMEM((w * KB, ROW), jnp.float32),
              pltpu.VMEM((sl,), jnp.float32),
              pltpu.SemaphoreType.DMA,
              pltpu.SemaphoreType.DMA,
          ]),
  )


@functools.cache
def _convi_kernel(w, R, KB, NPAD):
  nchunk = R // KB
  rounds = _cdiv(nchunk, NW)
  sl = NPAD // NS

  def body(src2d, dst2d, u2d, z2d, out, utab, acc,
           idxs, idxd, val, bounce, semg, sems):
    c = lax.axis_index("c")
    s = lax.axis_index("s")
    w32 = c * NS + s
    tsl = pl.ds(s * sl, sl)
    pltpu.sync_copy(z2d, bounce)
    pltpu.sync_copy(bounce, acc.at[tsl])
    pltpu.sync_copy(u2d.at[tsl], bounce)
    pltpu.sync_copy(bounce, utab.at[tsl])
    plsc.subcore_barrier()

    def round_body(k, _):
      cid = w32 + k * NW

      @pl.when(cid < nchunk)
      def _():
        csl = pl.ds(cid * KB, KB)
        pltpu.sync_copy(src2d.at[csl], idxs)
        pltpu.sync_copy(dst2d.at[csl], idxd)
        descs = [pltpu.async_copy(utab.at[idxs.at[j]], val.at[j], semg)
                 for j in range(KB)]
        for d in descs:
          d.wait()
        descs = [pltpu.async_copy(val.at[j], acc.at[idxd.at[j]], sems,
                                  add=True)
                 for j in range(KB)]
        for d in descs:
          d.wait()
      return 0

    lax.fori_loop(0, rounds, round_body, 0)
    plsc.subcore_barrier()
    pltpu.sync_copy(acc.at[tsl], bounce)
    pltpu.sync_copy(bounce, out.at[pl.ds(c * NPAD + s * sl, sl)])

  return pl.kernel(
      body,
      out_type=jax.ShapeDtypeStruct((NC * NPAD, w), jnp.float32),
      mesh=_mesh(),
      compiler_params=_PARAMS,
      scratch_types=[
          pltpu.VMEM_SHARED((NPAD, w), jnp.float32),
          pltpu.VMEM_SHARED((NPAD, w), jnp.float32),
          pltpu.VMEM((KB, ROW), jnp.int32),
          pltpu.VMEM((KB, ROW), jnp.int32),
          pltpu.VMEM((KB, ROW, w), jnp.float32),
          pltpu.VMEM((sl, w), jnp.float32),
          pltpu.SemaphoreType.DMA,
          pltpu.SemaphoreType.DMA,
      ],
  )


@functools.cache
def _readouti_kernel(RH, KB, NPAD, EH):
  nchunk = RH // KB
  rounds = _cdiv(nchunk, NW)
  cb = KB * ROW
  sl = NPAD // NS

  def body(sa2d, da2d, sb2d, db2d, h2d, out,
           htab, isa, ida, isb, idb, gb, prob, bounce, semg):
    c = lax.axis_index("c")
    s = lax.axis_index("s")
    w32 = c * NS + s
    tsl = pl.ds(s * sl, sl)
    pltpu.sync_copy(h2d.at[tsl], bounce)
    pltpu.sync_copy(bounce, htab.at[tsl])
    plsc.subcore_barrier()
    iota = lax.iota(jnp.int32, LN)

    def round_body(k, _):
      cid = w32 + k * NW

      @pl.when(cid < nchunk)
      def _():
        csl = pl.ds(cid * KB, KB)
        pltpu.sync_copy(sa2d.at[csl], isa)
        pltpu.sync_copy(da2d.at[csl], ida)
        pltpu.sync_copy(sb2d.at[csl], isb)
        pltpu.sync_copy(db2d.at[csl], idb)
        idrefs = (isa, ida, isb, idb)
        descs = [pltpu.async_copy(htab.at[idrefs[t].at[j]], gb.at[t, j],
                                  semg)
                 for j in range(KB) for t in range(4)]
        for d in descs:
          d.wait()

        def row(j, _):
          jv = jnp.full((LN,), j, jnp.int32)
          for i in range(ROW // LN):
            rowv = iota + (i * LN)
            accv = jnp.zeros((LN,), jnp.float32)
            for c4 in range(4):
              cv = jnp.full((LN,), c4, jnp.int32)
              va = plsc.load_gather(gb, [jnp.zeros((LN,), jnp.int32), jv,
                                         rowv, cv])
              vb = plsc.load_gather(gb, [jnp.full((LN,), 1, jnp.int32), jv,
                                         rowv, cv])
              accv = accv + va * vb
              vc = plsc.load_gather(gb, [jnp.full((LN,), 2, jnp.int32), jv,
                                         rowv, cv])
              vd = plsc.load_gather(gb, [jnp.full((LN,), 3, jnp.int32), jv,
                                         rowv, cv])
              accv = accv + vc * vd
            sv = accv * 0.5
            pv = 1.0 / (1.0 + jnp.exp(-sv))
            prob[pl.ds(j * ROW + i * LN, LN)] = pv
          return 0

        lax.fori_loop(0, KB, row, 0)
        pltpu.sync_copy(prob, out.at[pl.ds(cid * cb, cb)])
      return 0

    lax.fori_loop(0, rounds, round_body, 0)

  return pl.kernel(
      body,
      out_type=jax.ShapeDtypeStruct((EH,), jnp.float32),
      mesh=_mesh(),
      compiler_params=_PARAMS,
      scratch_types=[
          pltpu.VMEM_SHARED((NPAD, 4), jnp.float32),
          pltpu.VMEM((KB, ROW), jnp.int32),
          pltpu.VMEM((KB, ROW), jnp.int32),
          pltpu.VMEM((KB, ROW), jnp.int32),
          pltpu.VMEM((KB, ROW), jnp.int32),
          pltpu.VMEM((4, KB, ROW, 4), jnp.float32),
          pltpu.VMEM((KB * ROW,), jnp.float32),
          pltpu.VMEM((sl, 4), jnp.float32),
          pltpu.SemaphoreType.DMA,
      ],
  )


@functools.cache
def _readout_kernel(RH, KB, NPAD, EH):
  nchunk = RH // KB
  rounds = _cdiv(nchunk, NW)
  cb = KB * ROW
  sl = NPAD // NS

  def body(sa2d, da2d, sb2d, db2d, h0, h1, h2, h3, out,
           t0, t1, t2, t3, isa, ida, isb, idb, gbuf, prob, bounce, semg):
    htab = (t0, t1, t2, t3)
    hs = (h0, h1, h2, h3)
    c = lax.axis_index("c")
    s = lax.axis_index("s")
    w32 = c * NS + s
    tsl = pl.ds(s * sl, sl)
    for cc in range(4):
      pltpu.sync_copy(hs[cc].at[tsl], bounce)
      pltpu.sync_copy(bounce, htab[cc].at[tsl])
    plsc.subcore_barrier()

    def round_body(k, _):
      cid = w32 + k * NW

      @pl.when(cid < nchunk)
      def _():
        csl = pl.ds(cid * KB, KB)
        pltpu.sync_copy(sa2d.at[csl], isa)
        pltpu.sync_copy(da2d.at[csl], ida)
        pltpu.sync_copy(sb2d.at[csl], isb)
        pltpu.sync_copy(db2d.at[csl], idb)
        idrefs = (isa, ida, isb, idb)
        descs = [
            pltpu.async_copy(htab[cc].at[idrefs[t].at[j]],
                             gbuf.at[(4 * cc + t) * KB + j], semg)
            for j in range(KB) for cc in range(4) for t in range(4)]
        for d in descs:
          d.wait()

        def row(j, _):
          for i in range(ROW // LN):
            o = i * LN
            acc = jnp.zeros((LN,), jnp.float32)
            for cc in range(4):
              acc = acc + (gbuf[(4 * cc + 0) * KB + j, pl.ds(o, LN)] *
                           gbuf[(4 * cc + 1) * KB + j, pl.ds(o, LN)])
              acc = acc + (gbuf[(4 * cc + 2) * KB + j, pl.ds(o, LN)] *
                           gbuf[(4 * cc + 3) * KB + j, pl.ds(o, LN)])
            sv = acc * 0.5
            pv = 1.0 / (1.0 + jnp.exp(-sv))
            prob[pl.ds(j * ROW + o, LN)] = pv
          return 0

        lax.fori_loop(0, KB, row, 0)
        pltpu.sync_copy(prob, out.at[pl.ds(cid * cb, cb)])
      return 0

    lax.fori_loop(0, rounds, round_body, 0)

  return pl.kernel(
      body,
      out_type=jax.ShapeDtypeStruct((EH,), jnp.float32),
      mesh=_mesh(),
      compiler_params=_PARAMS,
      scratch_types=(
          [pltpu.VMEM_SHARED((NPAD,), jnp.float32)] * 4 + [
              pltpu.VMEM((KB, ROW), jnp.int32),
              pltpu.VMEM((KB, ROW), jnp.int32),
              pltpu.VMEM((KB, ROW), jnp.int32),
              pltpu.VMEM((KB, ROW), jnp.int32),
              pltpu.VMEM((16 * KB, ROW), jnp.float32),
              pltpu.VMEM((KB * ROW,), jnp.float32),
              pltpu.VMEM((sl,), jnp.float32),
              pltpu.SemaphoreType.DMA,
          ]),
  )


def _pad(col, npad):
  n = col.shape[0]
  return jnp.concatenate([col, jnp.zeros((npad - n,), col.dtype)])


def kernel(x, edge_index, W1, b1, W2, b2, W3, b3):
  n = x.shape[0]
  e = edge_index.shape[1]
  eh = e // 2
  npad = _cdiv(n, NS * LN) * NS * LN
  r = e // ROW
  rh = eh // ROW

  src0 = edge_index[0]
  dst0 = edge_index[1]
  src2d = src0.reshape(r, ROW)
  dst2d = dst0.reshape(r, ROW)
  sa2d = src0[:eh].reshape(rh, ROW)
  da2d = dst0[:eh].reshape(rh, ROW)
  sb2d = src0[eh:].reshape(rh, ROW)
  db2d = dst0[eh:].reshape(rh, ROW)

  degp = _degree_kernel(r, 16, npad)(dst2d).reshape(NC, npad)
  deg = degp[0, :n] + degp[1, :n] + 1.0
  dinv = lax.rsqrt(deg)

  sl = npad // NS

  def stack_pad(cols):
    return jnp.stack([_pad(u, npad) for u in cols], axis=-1)

  hw1 = x[:, 0] * W1[0, 0] + x[:, 1] * W1[1, 0] \
      + x[:, 2] * W1[2, 0] + x[:, 3] * W1[3, 0]
  u1 = dinv * hw1
  z1 = jnp.zeros((sl, 1), jnp.float32)
  p1 = _convi_kernel(1, r, 16, npad)(src2d, dst2d, stack_pad([u1]), z1)
  p1 = p1.reshape(NC, npad)
  h1 = jax.nn.relu(dinv * (p1[0, :n] + p1[1, :n] + u1) + b1[0])

  u2 = [dinv * (h1 * W2[0, cc]) for cc in range(2)]
  z2 = jnp.zeros((sl, 2), jnp.float32)
  p2 = _convi_kernel(2, r, 16, npad)(src2d, dst2d, stack_pad(u2), z2)
  p2 = p2.reshape(NC, npad, 2)
  h2 = [jax.nn.relu(dinv * (p2[0, :n, cc] + p2[1, :n, cc] + u2[cc]) + b2[cc])
        for cc in range(2)]

  u3 = [dinv * (h2[0] * W3[0, cc] + h2[1] * W3[1, cc]) for cc in range(4)]
  z4 = jnp.zeros((sl, 4), jnp.float32)
  p3 = _convi_kernel(4, r, 16, npad)(src2d, dst2d, stack_pad(u3), z4)
  p3 = p3.reshape(NC, npad, 4)
  h3 = [dinv * (p3[0, :n, cc] + p3[1, :n, cc] + u3[cc]) + b3[cc]
        for cc in range(4)]

  probs = _readouti_kernel(rh, 8, npad, eh)(
      sa2d, da2d, sb2d, db2d, stack_pad(h3))
  return probs[:, None]


# speedup vs baseline: 2.7847x; 2.7847x over previous
import functools

import jax
import jax.numpy as jnp
from jax import lax
from jax.experimental import pallas as pl
from jax.experimental.pallas import tpu as pltpu
from jax.experimental.pallas import tpu_sc as plsc

NC = 2
NS = 16
NW = NC * NS
LN = 16
ROW = 128


def _mesh():
  return plsc.VectorSubcoreMesh(
      core_axis_name="c", subcore_axis_name="s",
      num_cores=NC, num_subcores=NS)


_PARAMS = pltpu.CompilerParams(use_tc_tiling_on_sc=False,
                               needs_layout_passes=False)


def _cdiv(a, b):
  return (a + b - 1) // b


def _fill(ref, n, value):
  v = jnp.full((LN,), value, ref.dtype)

  def body(i, _):
    ref[pl.ds(i * LN, LN)] = v
    return 0

  lax.fori_loop(0, n // LN, body, 0)


def _interleave_col(flat_ref, d2_ref, n, cc):
  iota = lax.iota(jnp.int32, LN)
  ccv = jnp.full((LN,), cc, jnp.int32)

  def body(i, _):
    base = i * LN
    plsc.store_scatter(d2_ref, [iota + base, ccv], flat_ref[pl.ds(base, LN)])
    return 0

  lax.fori_loop(0, n // LN, body, 0)


def _zero2d(d2_ref, n, w):
  iota = lax.iota(jnp.int32, LN)
  zv = jnp.zeros((LN,), jnp.float32)
  for cc in range(w):
    ccv = jnp.full((LN,), cc, jnp.int32)

    def body(i, _):
      plsc.store_scatter(d2_ref, [iota + i * LN, ccv], zv)
      return 0

    lax.fori_loop(0, n // LN, body, 0)


def _deinterleave_col(d2_ref, flat_ref, n, cc):
  iota = lax.iota(jnp.int32, LN)
  ccv = jnp.full((LN,), cc, jnp.int32)

  def body(i, _):
    base = i * LN
    flat_ref[pl.ds(base, LN)] = plsc.load_gather(d2_ref, [iota + base, ccv])
    return 0

  lax.fori_loop(0, n // LN, body, 0)


@functools.cache
def _degree_kernel(R, KB, NPAD):
  nchunk = R // KB
  rounds = 4 * _cdiv(nchunk, 4 * NW)
  sl = NPAD // NS

  def body(dst2d, out, acc, idx, ones, zbuf, semi0, semi1, sems0, sems1):
    semi = (semi0, semi1)
    sems = (sems0, sems1)
    c = lax.axis_index("c")
    s = lax.axis_index("s")
    w32 = c * NS + s
    _fill(ones, ROW, 1.0)
    _fill(zbuf, sl, 0.0)
    pltpu.sync_copy(zbuf, acc.at[pl.ds(s * sl, sl)])
    plsc.subcore_barrier()

    def valid(m):
      return (m >= 0) & (m < rounds) & (w32 + m * NW < nchunk)

    def cslice(m):
      return pl.ds((w32 + m * NW) * KB, KB)

    def fire_load(m, slot, sem):
      @pl.when(valid(m))
      def _():
        pltpu.async_copy(dst2d.at[cslice(m)], idx.at[slot], sem)

    def drain_load(m, slot, sem):
      @pl.when(valid(m))
      def _():
        pltpu.make_async_copy(dst2d.at[cslice(m)], idx.at[slot], sem).wait()

    def fire_scats(m, slot, sem):
      @pl.when(valid(m))
      def _():
        for j in range(KB):
          pltpu.async_copy(ones, acc.at[idx.at[slot, j]], sem, add=True)

    def drain_scats(m, slot, sem):
      @pl.when(valid(m))
      def _():
        for j in range(KB):
          pltpu.make_async_copy(ones, acc.at[idx.at[slot, j]], sem).wait()

    fire_load(0, 0, semi[0])
    fire_load(1, 1, semi[1])

    def round_body(ko, _):
      for q in range(4):
        k = 4 * ko + q
        p = q % 2
        drain_scats(k - 2, (q + 2) % 4, sems[p])
        drain_load(k, q, semi[p])
        fire_scats(k, q, sems[p])
        fire_load(k + 2, (q + 2) % 4, semi[p])
      return 0

    lax.fori_loop(0, rounds // 4, round_body, 0)
    drain_scats(rounds - 2, 2, sems[0])
    drain_scats(rounds - 1, 3, sems[1])
    plsc.subcore_barrier()
    pltpu.sync_copy(acc.at[pl.ds(s * sl, sl)], zbuf)
    pltpu.sync_copy(zbuf, out.at[pl.ds(c * NPAD + s * sl, sl)])

  return pl.kernel(
      body,
      out_type=jax.ShapeDtypeStruct((NC * NPAD,), jnp.float32),
      mesh=_mesh(),
      compiler_params=_PARAMS,
      scratch_types=[
          pltpu.VMEM_SHARED((NPAD,), jnp.float32),
          pltpu.VMEM((4, KB, ROW), jnp.int32),
          pltpu.VMEM((ROW,), jnp.float32),
          pltpu.VMEM((sl,), jnp.float32),
          pltpu.SemaphoreType.DMA,
          pltpu.SemaphoreType.DMA,
          pltpu.SemaphoreType.DMA,
          pltpu.SemaphoreType.DMA,
      ],
  )


def _stage_chunk(sl):
  for d in (17, 23, 47, 2, 1):
    if sl % d == 0 and (sl // d) % 16 == 0:
      return sl // d
  return sl


@functools.cache
def _convp_kernel(w, R, KB, NPAD):
  nchunk = R // KB
  rounds = 4 * _cdiv(nchunk, 4 * NW)
  sl = NPAD // NS
  ch = _stage_chunk(sl)

  def body(src2d, dst2d, *rest):
    us = rest[:w]
    outs = rest[w:2 * w]
    (utab, acc, idxs, idxd, val, fb, b2,
     semi0, semi1, semg, sems0, sems1) = rest[2 * w:]
    semi = (semi0, semi1)
    sems = (sems0, sems1)
    c = lax.axis_index("c")
    s = lax.axis_index("s")
    w32 = c * NS + s
    tsl = pl.ds(s * sl, sl)
    if w == 1:
      _fill(fb, sl, 0.0)
      pltpu.sync_copy(fb, acc.at[tsl])
      pltpu.sync_copy(us[0].at[tsl], fb)
      pltpu.sync_copy(fb, utab.at[tsl])
    else:
      _zero2d(b2, ch, w)
      for q in range(sl // ch):
        qsl = pl.ds(s * sl + q * ch, ch)
        pltpu.sync_copy(b2, acc.at[qsl])
      for q in range(sl // ch):
        qsl = pl.ds(s * sl + q * ch, ch)
        for cc in range(w):
          pltpu.sync_copy(us[cc].at[qsl], fb)
          _interleave_col(fb, b2, ch, cc)
        pltpu.sync_copy(b2, utab.at[qsl])
    plsc.subcore_barrier()

    def valid(m):
      return (m >= 0) & (m < rounds) & (w32 + m * NW < nchunk)

    def cslice(m):
      return pl.ds((w32 + m * NW) * KB, KB)

    def fire_load(m, slot, sem):
      @pl.when(valid(m))
      def _():
        pltpu.async_copy(src2d.at[cslice(m)], idxs.at[slot], sem)
        pltpu.async_copy(dst2d.at[cslice(m)], idxd.at[slot], sem)

    def drain_load(m, slot, sem):
      @pl.when(valid(m))
      def _():
        pltpu.make_async_copy(src2d.at[cslice(m)], idxs.at[slot], sem).wait()
        pltpu.make_async_copy(dst2d.at[cslice(m)], idxd.at[slot], sem).wait()

    def fire_gathers(m, slot):
      @pl.when(valid(m))
      def _():
        for j in range(KB):
          pltpu.async_copy(utab.at[idxs.at[slot, j]], val.at[slot % 2, j],
                           semg)

    def drain_gathers(m, slot):
      @pl.when(valid(m))
      def _():
        for j in range(KB):
          pltpu.make_async_copy(utab.at[idxs.at[slot, j]],
                                val.at[slot % 2, j], semg).wait()

    def fire_scats(m, slot, sem):
      @pl.when(valid(m))
      def _():
        for j in range(KB):
          pltpu.async_copy(val.at[slot % 2, j], acc.at[idxd.at[slot, j]],
                           sem, add=True)

    def drain_scats(m, slot, sem):
      @pl.when(valid(m))
      def _():
        for j in range(KB):
          pltpu.make_async_copy(val.at[slot % 2, j],
                                acc.at[idxd.at[slot, j]], sem).wait()

    fire_load(0, 0, semi[0])
    fire_load(1, 1, semi[1])

    def round_body(ko, _):
      for q in range(4):
        k = 4 * ko + q
        p = q % 2
        drain_scats(k - 2, (q + 2) % 4, sems[p])
        drain_load(k, q, semi[p])
        fire_gathers(k, q)
        fire_load(k + 2, (q + 2) % 4, semi[p])
        drain_gathers(k, q)
        fire_scats(k, q, sems[p])
      return 0

    lax.fori_loop(0, rounds // 4, round_body, 0)
    drain_scats(rounds - 2, 2, sems[0])
    drain_scats(rounds - 1, 3, sems[1])
    plsc.subcore_barrier()
    if w == 1:
      pltpu.sync_copy(acc.at[tsl], fb)
      pltpu.sync_copy(fb, outs[0].at[pl.ds(c * NPAD + s * sl, sl)])
    else:
      for q in range(sl // ch):
        pltpu.sync_copy(acc.at[pl.ds(s * sl + q * ch, ch)], b2)
        for cc in range(w):
          _deinterleave_col(b2, fb, ch, cc)
          pltpu.sync_copy(
              fb, outs[cc].at[pl.ds(c * NPAD + s * sl + q * ch, ch)])

  tab_t = (pltpu.VMEM_SHARED((NPAD,), jnp.float32) if w == 1
           else pltpu.VMEM_SHARED((NPAD, w), jnp.float32))
  val_t = (pltpu.VMEM((2, KB, ROW), jnp.float32) if w == 1
           else pltpu.VMEM((2, KB, ROW, w), jnp.float32))
  fb_len = sl if w == 1 else ch
  return pl.kernel(
      body,
      out_type=[jax.ShapeDtypeStruct((NC * NPAD,), jnp.float32)] * w,
      mesh=_mesh(),
      compiler_params=_PARAMS,
      scratch_types=[
          tab_t,
          tab_t,
          pltpu.VMEM((4, KB, ROW), jnp.int32),
          pltpu.VMEM((4, KB, ROW), jnp.int32),
          val_t,
          pltpu.VMEM((fb_len,), jnp.float32),
          pltpu.VMEM((ch, max(w, 2)), jnp.float32),
          pltpu.SemaphoreType.DMA,
          pltpu.SemaphoreType.DMA,
          pltpu.SemaphoreType.DMA,
          pltpu.SemaphoreType.DMA,
          pltpu.SemaphoreType.DMA,
      ],
  )


@functools.cache
def _readoutp_kernel(RH, KB, NPAD, EH):
  nchunk = RH // KB
  rounds = 4 * _cdiv(nchunk, 4 * NW)
  cb = KB * ROW
  sl = NPAD // NS
  ch = _stage_chunk(sl)

  def body(sa2d, da2d, sb2d, db2d, h0, h1, h2, h3, out,
           htab, idx, gb, prob, fb, b2, semi0, semi1, semg0, semg1):
    hs = (h0, h1, h2, h3)
    semi = (semi0, semi1)
    semg = (semg0, semg1)
    id2d = (sa2d, da2d, sb2d, db2d)
    c = lax.axis_index("c")
    s = lax.axis_index("s")
    w32 = c * NS + s
    for q in range(sl // ch):
      qsl = pl.ds(s * sl + q * ch, ch)
      for cc in range(4):
        pltpu.sync_copy(hs[cc].at[qsl], fb)
        _interleave_col(fb, b2, ch, cc)
      pltpu.sync_copy(b2, htab.at[qsl])
    plsc.subcore_barrier()
    iota = lax.iota(jnp.int32, LN)

    def valid(m):
      return (m >= 0) & (m < rounds) & (w32 + m * NW < nchunk)

    def cslice(m):
      return pl.ds((w32 + m * NW) * KB, KB)

    def fire_load(m, slot, sem):
      @pl.when(valid(m))
      def _():
        for t in range(4):
          pltpu.async_copy(id2d[t].at[cslice(m)], idx.at[slot, t], sem)

    def drain_load(m, slot, sem):
      @pl.when(valid(m))
      def _():
        for t in range(4):
          pltpu.make_async_copy(id2d[t].at[cslice(m)], idx.at[slot, t],
                                sem).wait()

    def fire_gathers(m, slot, sem):
      @pl.when(valid(m))
      def _():
        for t in range(4):
          for j in range(KB):
            pltpu.async_copy(htab.at[idx.at[slot, t, j]],
                             gb.at[slot % 2, t, j], sem)

    def drain_gathers(m, slot, sem):
      @pl.when(valid(m))
      def _():
        for t in range(4):
          for j in range(KB):
            pltpu.make_async_copy(htab.at[idx.at[slot, t, j]],
                                  gb.at[slot % 2, t, j], sem).wait()

    def compute(m, b):
      @pl.when(valid(m))
      def _():

        def row(j, _):
          jv = jnp.full((LN,), j, jnp.int32)
          bv = jnp.full((LN,), b, jnp.int32)
          for i in range(ROW // LN):
            rowv = iota + (i * LN)
            accv = jnp.zeros((LN,), jnp.float32)
            for c4 in range(4):
              cv = jnp.full((LN,), c4, jnp.int32)
              va = plsc.load_gather(
                  gb, [bv, jnp.zeros((LN,), jnp.int32), jv, rowv, cv])
              vb = plsc.load_gather(
                  gb, [bv, jnp.full((LN,), 1, jnp.int32), jv, rowv, cv])
              accv = accv + va * vb
              vc = plsc.load_gather(
                  gb, [bv, jnp.full((LN,), 2, jnp.int32), jv, rowv, cv])
              vd = plsc.load_gather(
                  gb, [bv, jnp.full((LN,), 3, jnp.int32), jv, rowv, cv])
              accv = accv + vc * vd
            sv = accv * 0.5
            pv = 1.0 / (1.0 + jnp.exp(-sv))
            prob[pl.ds(j * ROW + i * LN, LN)] = pv
          return 0

        lax.fori_loop(0, KB, row, 0)
        pltpu.sync_copy(prob, out.at[pl.ds((w32 + m * NW) * cb, cb)])

      return None

    fire_load(0, 0, semi[0])
    fire_load(1, 1, semi[1])

    def round_body(ko, _):
      for q in range(4):
        k = 4 * ko + q
        p = q % 2
        drain_load(k, q, semi[p])
        fire_gathers(k, q, semg[p])
        fire_load(k + 2, (q + 2) % 4, semi[p])
        drain_gathers(k - 1, (q + 3) % 4, semg[1 - p])
        compute(k - 1, (q + 1) % 2)
      return 0

    lax.fori_loop(0, rounds // 4, round_body, 0)
    drain_gathers(rounds - 1, 3, semg[1])
    compute(rounds - 1, 1)

  return pl.kernel(
      body,
      out_type=jax.ShapeDtypeStruct((EH,), jnp.float32),
      mesh=_mesh(),
      compiler_params=_PARAMS,
      scratch_types=[
          pltpu.VMEM_SHARED((NPAD, 4), jnp.float32),
          pltpu.VMEM((4, 4, KB, ROW), jnp.int32),
          pltpu.VMEM((2, 4, KB, ROW, 4), jnp.float32),
          pltpu.VMEM((KB * ROW,), jnp.float32),
          pltpu.VMEM((ch,), jnp.float32),
          pltpu.VMEM((ch, 4), jnp.float32),
          pltpu.SemaphoreType.DMA,
          pltpu.SemaphoreType.DMA,
          pltpu.SemaphoreType.DMA,
          pltpu.SemaphoreType.DMA,
      ],
  )


def _pad(col, npad):
  n = col.shape[0]
  return jnp.concatenate([col, jnp.zeros((npad - n,), col.dtype)])


def kernel(x, edge_index, W1, b1, W2, b2, W3, b3):
  n = x.shape[0]
  e = edge_index.shape[1]
  eh = e // 2
  npad = _cdiv(n, NS * LN) * NS * LN
  r = e // ROW
  rh = eh // ROW

  src0 = edge_index[0]
  dst0 = edge_index[1]
  src2d = src0.reshape(r, ROW)
  dst2d = dst0.reshape(r, ROW)
  sa2d = src0[:eh].reshape(rh, ROW)
  da2d = dst0[:eh].reshape(rh, ROW)
  sb2d = src0[eh:].reshape(rh, ROW)
  db2d = dst0[eh:].reshape(rh, ROW)

  degp = _degree_kernel(r, 16, npad)(dst2d).reshape(NC, npad)
  deg = degp[0, :n] + degp[1, :n] + 1.0
  dinv = lax.rsqrt(deg)

  hw1 = x[:, 0] * W1[0, 0] + x[:, 1] * W1[1, 0] \
      + x[:, 2] * W1[2, 0] + x[:, 3] * W1[3, 0]
  u1 = dinv * hw1
  (p1,) = _convp_kernel(1, r, 16, npad)(src2d, dst2d, _pad(u1, npad))
  p1 = p1.reshape(NC, npad)
  h1 = jax.nn.relu(dinv * (p1[0, :n] + p1[1, :n] + u1) + b1[0])

  u2 = [dinv * (h1 * W2[0, cc]) for cc in range(2)]
  p2 = _convp_kernel(2, r, 16, npad)(
      src2d, dst2d, *[_pad(u, npad) for u in u2])
  p2 = [p.reshape(NC, npad) for p in p2]
  h2 = [jax.nn.relu(dinv * (p2[cc][0, :n] + p2[cc][1, :n] + u2[cc]) + b2[cc])
        for cc in range(2)]

  u3 = [dinv * (h2[0] * W3[0, cc] + h2[1] * W3[1, cc]) for cc in range(4)]
  p3 = _convp_kernel(4, r, 16, npad)(
      src2d, dst2d, *[_pad(u, npad) for u in u3])
  p3 = [p.reshape(NC, npad) for p in p3]
  h3 = [dinv * (p3[cc][0, :n] + p3[cc][1, :n] + u3[cc]) + b3[cc]
        for cc in range(4)]

  probs = _readoutp_kernel(rh, 8, npad, eh)(
      sa2d, da2d, sb2d, db2d, *[_pad(h, npad) for h in h3])
  return probs[:, None]
